# baseline (speedup 1.0000x reference)
import jax
import jax.numpy as jnp
from jax import lax
from jax.experimental import pallas as pl
from jax.experimental.pallas import tpu as pltpu

EMB = 1024
INTERMED = 1024
N_EXPERTS = 8
N_SHARED = 2
N_ROUTED = 6
N_TOK = 4096
TT = 1024
NT = N_TOK // TT


def _router_body(x_ref, rw_ref, rb_ref, gd_ref):
    x = x_ref[...]
    logits = lax.dot_general(x, rw_ref[...], (((1,), (1,)), ((), ())),
                             preferred_element_type=jnp.float32)
    logits = logits + rb_ref[...]
    col = lax.broadcasted_iota(jnp.int32, (N_TOK, 128), 1)
    valid = col < N_ROUTED
    lm = jnp.where(valid, logits, jnp.float32(-1e30))
    m = jnp.max(lm, axis=1, keepdims=True)
    p = jnp.where(valid, jnp.exp(lm - m), 0.0)
    aff = p / jnp.sum(p, axis=1, keepdims=True)
    v1 = jnp.max(aff, axis=1, keepdims=True)
    i1 = jnp.min(jnp.where((aff == v1) & valid, col, 127), axis=1, keepdims=True)
    rem = valid & (col != i1)
    affr = jnp.where(rem, aff, -1.0)
    v2 = jnp.max(affr, axis=1, keepdims=True)
    i2 = jnp.min(jnp.where((affr == v2) & rem, col, 127), axis=1, keepdims=True)
    routed_col = col - N_SHARED
    g_routed = (jnp.where(i1 == routed_col, v1, 0.0)
                + jnp.where(i2 == routed_col, v2, 0.0))
    gd_ref[...] = jnp.where(col < N_SHARED, 1.0,
                            jnp.where(col < N_EXPERTS, g_routed, 0.0)
                            ).astype(jnp.bfloat16)


def _gelu(h):
    return 0.5 * h * (1.0 + lax.erf(h * jnp.float32(0.7071067811865476)))


def _moe_body(xres_ref, xb_ref, g_ref, sw1_ref, rw1_ref, sb1_ref, rb1_ref,
              sw2_ref, rw2_ref, sb2_ref, rb2_ref, out_ref):
    e = pl.program_id(1)
    col = lax.broadcasted_iota(jnp.int32, (TT, 128), 1)
    g = jnp.sum(jnp.where(col == e, g_ref[...].astype(jnp.float32), 0.0),
                axis=1, keepdims=True)
    xb = xb_ref[...]

    @pl.when(e < N_SHARED)
    def _shared():
        h = _gelu(lax.dot_general(xb, sw1_ref[0], (((1,), (1,)), ((), ())),
                                  preferred_element_type=jnp.float32)
                  + sb1_ref[0])
        y = lax.dot_general(h.astype(jnp.bfloat16), sw2_ref[0],
                            (((1,), (1,)), ((), ())),
                            preferred_element_type=jnp.float32) + sb2_ref[0]

        @pl.when(e == 0)
        def _():
            out_ref[...] = xres_ref[...] + g * y

        @pl.when(e > 0)
        def _():
            out_ref[...] += g * y

    @pl.when(e >= N_SHARED)
    def _routed():
        h = _gelu(lax.dot_general(xb, rw1_ref[0], (((1,), (1,)), ((), ())),
                                  preferred_element_type=jnp.float32)
                  + rb1_ref[0])
        y = lax.dot_general(h.astype(jnp.bfloat16), rw2_ref[0],
                            (((1,), (1,)), ((), ())),
                            preferred_element_type=jnp.float32) + rb2_ref[0]
        out_ref[...] += g * y


@jax.jit
def _moe(x2d, xb2d, rw_p, rb_p, sw1, sb1, sw2, sb2, rw1, rb1, rw2, rb2):
    gd = pl.pallas_call(
        _router_body,
        out_shape=jax.ShapeDtypeStruct((N_TOK, 128), jnp.bfloat16),
    )(x2d, rw_p, rb_p)

    def _sh(e):
        return jnp.minimum(e, N_SHARED - 1)

    def _rt(e):
        return jnp.maximum(e - N_SHARED, 0)

    return pl.pallas_call(
        _moe_body,
        grid=(NT, N_EXPERTS),
        in_specs=[
            pl.BlockSpec((TT, EMB),
                         lambda t, e: (jnp.where(e == 0, t, 0), 0)),
            pl.BlockSpec((TT, EMB), lambda t, e: (t, 0)),
            pl.BlockSpec((TT, 128), lambda t, e: (t, 0)),
            pl.BlockSpec((1, INTERMED, EMB), lambda t, e: (_sh(e), 0, 0)),
            pl.BlockSpec((1, INTERMED, EMB), lambda t, e: (_rt(e), 0, 0)),
            pl.BlockSpec((1, 1, INTERMED), lambda t, e: (_sh(e), 0, 0)),
            pl.BlockSpec((1, 1, INTERMED), lambda t, e: (_rt(e), 0, 0)),
            pl.BlockSpec((1, EMB, INTERMED), lambda t, e: (_sh(e), 0, 0)),
            pl.BlockSpec((1, EMB, INTERMED), lambda t, e: (_rt(e), 0, 0)),
            pl.BlockSpec((1, 1, EMB), lambda t, e: (_sh(e), 0, 0)),
            pl.BlockSpec((1, 1, EMB), lambda t, e: (_rt(e), 0, 0)),
        ],
        out_specs=pl.BlockSpec((TT, EMB), lambda t, e: (t, 0)),
        out_shape=jax.ShapeDtypeStruct((N_TOK, EMB), jnp.float32),
        compiler_params=pltpu.CompilerParams(
            dimension_semantics=("arbitrary", "arbitrary")),
    )(x2d, xb2d, gd, sw1, rw1, sb1, rb1, sw2, rw2, sb2, rb2)


def kernel(x, router_W, router_b, shared_W1, shared_b1, shared_W2, shared_b2,
           routed_W1, routed_b1, routed_W2, routed_b2):
    B, S, _ = x.shape
    x2d = x.reshape(B * S, EMB)
    rw_p = jnp.zeros((128, EMB), jnp.float32).at[:N_ROUTED].set(router_W)
    rb_p = jnp.zeros((1, 128), jnp.float32).at[0, :N_ROUTED].set(router_b)
    bf = jnp.bfloat16
    out = _moe(x2d, x2d.astype(bf), rw_p, rb_p,
               shared_W1.astype(bf), shared_b1[:, None, :],
               shared_W2.astype(bf), shared_b2[:, None, :],
               routed_W1.astype(bf), routed_b1[:, None, :],
               routed_W2.astype(bf), routed_b2[:, None, :])
    return out.reshape(B, S, EMB)


# speedup vs baseline: 1.1498x; 1.1498x over previous
import jax
import jax.numpy as jnp
from jax import lax
from jax.experimental import pallas as pl
from jax.experimental.pallas import tpu as pltpu

EMB = 1024
INTERMED = 1024
N_EXPERTS = 8
N_SHARED = 2
N_ROUTED = 6
N_TOK = 4096
TT = 1024
NT = N_TOK // TT


def _router_body(x_ref, rw_ref, rb_ref, gd_ref):
    x = x_ref[...]
    logits = lax.dot_general(x, rw_ref[...], (((1,), (1,)), ((), ())),
                             preferred_element_type=jnp.float32)
    logits = logits + rb_ref[...]
    col = lax.broadcasted_iota(jnp.int32, (N_TOK, 128), 1)
    valid = col < N_ROUTED
    lm = jnp.where(valid, logits, jnp.float32(-1e30))
    m = jnp.max(lm, axis=1, keepdims=True)
    p = jnp.where(valid, jnp.exp(lm - m), 0.0)
    aff = p / jnp.sum(p, axis=1, keepdims=True)
    v1 = jnp.max(aff, axis=1, keepdims=True)
    i1 = jnp.min(jnp.where((aff == v1) & valid, col, 127), axis=1, keepdims=True)
    rem = valid & (col != i1)
    affr = jnp.where(rem, aff, -1.0)
    v2 = jnp.max(affr, axis=1, keepdims=True)
    i2 = jnp.min(jnp.where((affr == v2) & rem, col, 127), axis=1, keepdims=True)
    routed_col = col - N_SHARED
    g_routed = (jnp.where(i1 == routed_col, v1, 0.0)
                + jnp.where(i2 == routed_col, v2, 0.0))
    gd_ref[...] = jnp.where(col < N_SHARED, 1.0,
                            jnp.where(col < N_EXPERTS, g_routed, 0.0)
                            ).astype(jnp.bfloat16)


def _gelu(h):
    return 0.5 * h * (1.0 + lax.erf(h * jnp.float32(0.7071067811865476)))


def _moe_body(x_ref, g_ref, sw1_ref, rw1_ref, sb1_ref, rb1_ref,
              sw2_ref, rw2_ref, sb2_ref, rb2_ref, out_ref):
    e = pl.program_id(1)
    col = lax.broadcasted_iota(jnp.int32, (TT, 128), 1)
    g = jnp.sum(jnp.where(col == e, g_ref[...].astype(jnp.float32), 0.0),
                axis=1, keepdims=True)
    x = x_ref[...]

    @pl.when(e < N_SHARED)
    def _shared():
        h = _gelu(lax.dot_general(x, sw1_ref[0], (((1,), (1,)), ((), ())),
                                  preferred_element_type=jnp.float32)
                  + sb1_ref[0])
        y = lax.dot_general(h, sw2_ref[0], (((1,), (1,)), ((), ())),
                            preferred_element_type=jnp.float32) + sb2_ref[0]

        @pl.when(e == 0)
        def _():
            out_ref[...] = x + g * y

        @pl.when(e > 0)
        def _():
            out_ref[...] += g * y

    @pl.when(e >= N_SHARED)
    def _routed():
        h = _gelu(lax.dot_general(x, rw1_ref[0], (((1,), (1,)), ((), ())),
                                  preferred_element_type=jnp.float32)
                  + rb1_ref[0])
        y = lax.dot_general(h, rw2_ref[0], (((1,), (1,)), ((), ())),
                            preferred_element_type=jnp.float32) + rb2_ref[0]
        out_ref[...] += g * y


@jax.jit
def _moe(x2d, rw_p, rb_p, sw1, sb1, sw2, sb2, rw1, rb1, rw2, rb2):
    gd = pl.pallas_call(
        _router_body,
        out_shape=jax.ShapeDtypeStruct((N_TOK, 128), jnp.bfloat16),
    )(x2d, rw_p, rb_p)

    def _sh(e):
        return jnp.minimum(e, N_SHARED - 1)

    def _rt(e):
        return jnp.maximum(e - N_SHARED, 0)

    return pl.pallas_call(
        _moe_body,
        grid=(NT, N_EXPERTS),
        in_specs=[
            pl.BlockSpec((TT, EMB), lambda t, e: (t, 0)),
            pl.BlockSpec((TT, 128), lambda t, e: (t, 0)),
            pl.BlockSpec((1, INTERMED, EMB), lambda t, e: (_sh(e), 0, 0)),
            pl.BlockSpec((1, INTERMED, EMB), lambda t, e: (_rt(e), 0, 0)),
            pl.BlockSpec((1, 1, INTERMED), lambda t, e: (_sh(e), 0, 0)),
            pl.BlockSpec((1, 1, INTERMED), lambda t, e: (_rt(e), 0, 0)),
            pl.BlockSpec((1, EMB, INTERMED), lambda t, e: (_sh(e), 0, 0)),
            pl.BlockSpec((1, EMB, INTERMED), lambda t, e: (_rt(e), 0, 0)),
            pl.BlockSpec((1, 1, EMB), lambda t, e: (_sh(e), 0, 0)),
            pl.BlockSpec((1, 1, EMB), lambda t, e: (_rt(e), 0, 0)),
        ],
        out_specs=pl.BlockSpec((TT, EMB), lambda t, e: (t, 0)),
        out_shape=jax.ShapeDtypeStruct((N_TOK, EMB), jnp.float32),
        compiler_params=pltpu.CompilerParams(
            dimension_semantics=("arbitrary", "arbitrary")),
    )(x2d, gd, sw1, rw1, sb1, rb1, sw2, rw2, sb2, rb2)


def kernel(x, router_W, router_b, shared_W1, shared_b1, shared_W2, shared_b2,
           routed_W1, routed_b1, routed_W2, routed_b2):
    B, S, _ = x.shape
    x2d = x.reshape(B * S, EMB)
    rw_p = jnp.zeros((128, EMB), jnp.float32).at[:N_ROUTED].set(router_W)
    rb_p = jnp.zeros((1, 128), jnp.float32).at[0, :N_ROUTED].set(router_b)
    out = _moe(x2d, rw_p, rb_p,
               shared_W1, shared_b1[:, None, :], shared_W2, shared_b2[:, None, :],
               routed_W1, routed_b1[:, None, :], routed_W2, routed_b2[:, None, :])
    return out.reshape(B, S, EMB)


# speedup vs baseline: 1.1502x; 1.0004x over previous
import jax
import jax.numpy as jnp
from jax import lax
from jax.experimental import pallas as pl
from jax.experimental.pallas import tpu as pltpu

EMB = 1024
INTERMED = 1024
N_EXPERTS = 8
N_SHARED = 2
N_ROUTED = 6
N_TOK = 4096
TT = 1024
NT = N_TOK // TT


def _router_body(x_ref, rw_ref, rb_ref, gd_ref):
    x = x_ref[...]
    logits = lax.dot_general(x, rw_ref[...], (((1,), (1,)), ((), ())),
                             preferred_element_type=jnp.float32)
    logits = logits + rb_ref[...]
    col = lax.broadcasted_iota(jnp.int32, (N_TOK, 128), 1)
    valid = col < N_ROUTED
    lm = jnp.where(valid, logits, jnp.float32(-1e30))
    m = jnp.max(lm, axis=1, keepdims=True)
    p = jnp.where(valid, jnp.exp(lm - m), 0.0)
    aff = p / jnp.sum(p, axis=1, keepdims=True)
    v1 = jnp.max(aff, axis=1, keepdims=True)
    i1 = jnp.min(jnp.where((aff == v1) & valid, col, 127), axis=1, keepdims=True)
    rem = valid & (col != i1)
    affr = jnp.where(rem, aff, -1.0)
    v2 = jnp.max(affr, axis=1, keepdims=True)
    i2 = jnp.min(jnp.where((affr == v2) & rem, col, 127), axis=1, keepdims=True)
    routed_col = col - N_SHARED
    g_routed = (jnp.where(i1 == routed_col, v1, 0.0)
                + jnp.where(i2 == routed_col, v2, 0.0))
    gd_ref[...] = jnp.where(col < N_SHARED, 1.0,
                            jnp.where(col < N_EXPERTS, g_routed, 0.0)
                            ).astype(jnp.bfloat16)


def _gelu(h):
    return 0.5 * h * (1.0 + lax.erf(h * jnp.float32(0.7071067811865476)))


def _moe_body(x_ref, g_ref, sw1_ref, rw1_ref, sb1_ref, rb1_ref,
              sw2_ref, rw2_ref, sb2_ref, rb2_ref, out_ref):
    e = pl.program_id(1)
    col = lax.broadcasted_iota(jnp.int32, (TT, 128), 1)
    g = jnp.sum(jnp.where(col == e, g_ref[...].astype(jnp.float32), 0.0),
                axis=1, keepdims=True)
    x = x_ref[...]

    @pl.when(e < N_SHARED)
    def _shared():
        h = _gelu(lax.dot_general(x, sw1_ref[0], (((1,), (1,)), ((), ())),
                                  preferred_element_type=jnp.float32)
                  + sb1_ref[0])
        y = lax.dot_general(h, sw2_ref[0], (((1,), (1,)), ((), ())),
                            preferred_element_type=jnp.float32) + sb2_ref[0]

        @pl.when(e == 0)
        def _():
            out_ref[...] = x + g * y

        @pl.when(e > 0)
        def _():
            out_ref[...] += g * y

    @pl.when(e >= N_SHARED)
    def _routed():
        h = _gelu(lax.dot_general(x, rw1_ref[0], (((1,), (1,)), ((), ())),
                                  preferred_element_type=jnp.float32)
                  + rb1_ref[0])
        y = lax.dot_general(h, rw2_ref[0], (((1,), (1,)), ((), ())),
                            preferred_element_type=jnp.float32) + rb2_ref[0]
        out_ref[...] += g * y


@jax.jit
def _moe(x2d, rw_p, rb_p, sw1, sb1, sw2, sb2, rw1, rb1, rw2, rb2):
    gd = pl.pallas_call(
        _router_body,
        out_shape=jax.ShapeDtypeStruct((N_TOK, 128), jnp.bfloat16),
    )(x2d, rw_p, rb_p)

    def _sh(e):
        return jnp.minimum(e, N_SHARED - 1)

    def _rt(e):
        return jnp.maximum(e - N_SHARED, 0)

    return pl.pallas_call(
        _moe_body,
        grid=(NT, N_EXPERTS),
        in_specs=[
            pl.BlockSpec((TT, EMB), lambda t, e: (t, 0)),
            pl.BlockSpec((TT, 128), lambda t, e: (t, 0)),
            pl.BlockSpec((1, INTERMED, EMB), lambda t, e: (_sh(e), 0, 0)),
            pl.BlockSpec((1, INTERMED, EMB), lambda t, e: (_rt(e), 0, 0)),
            pl.BlockSpec((1, 1, INTERMED), lambda t, e: (_sh(e), 0, 0)),
            pl.BlockSpec((1, 1, INTERMED), lambda t, e: (_rt(e), 0, 0)),
            pl.BlockSpec((1, EMB, INTERMED), lambda t, e: (_sh(e), 0, 0)),
            pl.BlockSpec((1, EMB, INTERMED), lambda t, e: (_rt(e), 0, 0)),
            pl.BlockSpec((1, 1, EMB), lambda t, e: (_sh(e), 0, 0)),
            pl.BlockSpec((1, 1, EMB), lambda t, e: (_rt(e), 0, 0)),
        ],
        out_specs=pl.BlockSpec((TT, EMB), lambda t, e: (t, 0)),
        out_shape=jax.ShapeDtypeStruct((N_TOK, EMB), jnp.float32),
        compiler_params=pltpu.CompilerParams(
            dimension_semantics=("parallel", "arbitrary")),
    )(x2d, gd, sw1, rw1, sb1, rb1, sw2, rw2, sb2, rb2)


def kernel(x, router_W, router_b, shared_W1, shared_b1, shared_W2, shared_b2,
           routed_W1, routed_b1, routed_W2, routed_b2):
    B, S, _ = x.shape
    x2d = x.reshape(B * S, EMB)
    rw_p = jnp.zeros((128, EMB), jnp.float32).at[:N_ROUTED].set(router_W)
    rb_p = jnp.zeros((1, 128), jnp.float32).at[0, :N_ROUTED].set(router_b)
    out = _moe(x2d, rw_p, rb_p,
               shared_W1, shared_b1[:, None, :], shared_W2, shared_b2[:, None, :],
               routed_W1, routed_b1[:, None, :], routed_W2, routed_b2[:, None, :])
    return out.reshape(B, S, EMB)
